# baseline (speedup 1.0000x reference)
import jax
import jax.numpy as jnp
from jax import lax
from jax.experimental import pallas as pl
from jax.experimental.pallas import tpu as pltpu
from jax.experimental.pallas import tpu_sc as plsc

_L = 16
_W = 3072
_NW = 32
_G = 8
_SCROWS = 8 * _G
_MAIN = _NW * _W
_REM = 1696
_TC_RING = 4


def _tanh16(v):
    t = jnp.exp(v + v)
    return 1.0 - 2.0 / (t + 1.0)


def _sc_body(x_hbm, o_hbm, ib0, ib1, ob0, ob1, is0, is1, os0, os1):
    wid = lax.axis_index("s") * 2 + lax.axis_index("c")
    col = pl.multiple_of(wid * _W, 128)
    ibufs, obufs = (ib0, ib1), (ob0, ob1)
    isems, osems = (is0, is1), (os0, os1)

    def in_cp(g, b):
        r0 = pl.multiple_of(g * 8, 8)
        return pltpu.make_async_copy(
            x_hbm.at[pl.ds(r0, 8), pl.ds(col, _W)], ibufs[b], isems[b])

    def out_cp(g, b):
        r0 = pl.multiple_of(g * 8, 8)
        return pltpu.make_async_copy(
            obufs[b], o_hbm.at[pl.ds(r0, 8), pl.ds(col, _W)], osems[b])

    def compute(b):
        @plsc.parallel_loop(0, _W // _L, 1, unroll=4)
        def _(j, b=b):
            c = pl.multiple_of(j * _L, _L)
            for r in range(8):
                obufs[b][r, pl.ds(c, _L)] = _tanh16(
                    ibufs[b][r, pl.ds(c, _L)])

    in_cp(0, 0).start()
    in_cp(1, 1).start()

    def step(s, _):
        for b in range(2):
            g = 2 * s + b
            in_cp(g, b).wait()

            @pl.when(s > 0)
            def _():
                out_cp(g - 2, b).wait()

            compute(b)
            out_cp(g, b).start()

            @pl.when(s < _G // 2 - 1)
            def _():
                in_cp(g + 2, b).start()
        return 0

    lax.fori_loop(0, _G // 2, step, 0, unroll=False)
    out_cp(_G - 2, 0).wait()
    out_cp(_G - 1, 1).wait()


def _tc_body(x_hbm, o_hbm, *scratch):
    ibufs = scratch[:_TC_RING]
    obufs = scratch[_TC_RING:2 * _TC_RING]
    isems = scratch[2 * _TC_RING]
    osems = scratch[2 * _TC_RING + 1]
    nchunk = o_hbm.shape[0] // 8

    def in_cp(c, s):
        return pltpu.make_async_copy(
            x_hbm.at[pl.ds(_SCROWS + c * 8, 8), :], ibufs[s], isems.at[s])

    def out_cp(c, s):
        return pltpu.make_async_copy(
            obufs[s], o_hbm.at[pl.ds(c * 8, 8), :], osems.at[s])

    for c in range(min(_TC_RING, nchunk)):
        in_cp(c, c).start()
    for c in range(nchunk):
        s = c % _TC_RING
        in_cp(c, s).wait()
        if c >= _TC_RING:
            out_cp(c - _TC_RING, s).wait()
        obufs[s][...] = jnp.tanh(ibufs[s][...])
        out_cp(c, s).start()
        nc = c + _TC_RING
        if nc < nchunk:
            in_cp(nc, s).start()
    for c in range(max(nchunk - _TC_RING, 0), nchunk):
        out_cp(c, c % _TC_RING).wait()


def _tail_body(o_in_hbm, x_hbm, o_hbm, buf, sem_i, sem_o):
    del o_in_hbm
    pltpu.make_async_copy(
        x_hbm.at[pl.ds(0, _SCROWS), pl.ds(_MAIN, _REM)], buf, sem_i).start()
    pltpu.make_async_copy(
        x_hbm.at[pl.ds(0, _SCROWS), pl.ds(_MAIN, _REM)], buf, sem_i).wait()
    buf[...] = jnp.tanh(buf[...])
    pltpu.make_async_copy(
        buf, o_hbm.at[:, pl.ds(_MAIN, _REM)], sem_o).start()
    pltpu.make_async_copy(
        buf, o_hbm.at[:, pl.ds(_MAIN, _REM)], sem_o).wait()


def kernel(mean, stddev):
    del stddev
    m, n = mean.shape
    mesh = plsc.VectorSubcoreMesh(core_axis_name="c", subcore_axis_name="s")
    sc = pl.kernel(
        _sc_body,
        out_type=jax.ShapeDtypeStruct((_SCROWS, n), jnp.float32),
        mesh=mesh,
        scratch_types=(
            [pltpu.VMEM((8, _W), jnp.float32) for _ in range(4)]
            + [pltpu.SemaphoreType.DMA for _ in range(4)]
        ),
    )
    sc_out = sc(mean)

    bottom = pl.pallas_call(
        _tc_body,
        in_specs=[pl.BlockSpec(memory_space=pl.ANY)],
        out_specs=pl.BlockSpec(memory_space=pl.ANY),
        out_shape=jax.ShapeDtypeStruct((m - _SCROWS, n), jnp.float32),
        scratch_shapes=(
            [pltpu.VMEM((8, n), jnp.float32) for _ in range(2 * _TC_RING)]
            + [pltpu.SemaphoreType.DMA((_TC_RING,)),
               pltpu.SemaphoreType.DMA((_TC_RING,))]
        ),
    )(mean)

    top = pl.pallas_call(
        _tail_body,
        in_specs=[pl.BlockSpec(memory_space=pl.ANY),
                  pl.BlockSpec(memory_space=pl.ANY)],
        out_specs=pl.BlockSpec(memory_space=pl.ANY),
        out_shape=jax.ShapeDtypeStruct((_SCROWS, n), jnp.float32),
        input_output_aliases={0: 0},
        scratch_shapes=[pltpu.VMEM((_SCROWS, _REM), jnp.float32),
                        pltpu.SemaphoreType.DMA,
                        pltpu.SemaphoreType.DMA],
    )(sc_out, mean)

    return jnp.concatenate([top, bottom], axis=0)


# speedup vs baseline: 1.1207x; 1.1207x over previous
import jax
import jax.numpy as jnp
from jax import lax
from jax.experimental import pallas as pl
from jax.experimental.pallas import tpu as pltpu
from jax.experimental.pallas import tpu_sc as plsc

_L = 16
_W = 3072
_NW = 32
_G = 8
_SCROWS = 8 * _G
_MAIN = _NW * _W
_REM = 1696
_TC_RING = 4


def _tanh16(v):
    t = jnp.exp(v + v)
    return 1.0 - 2.0 / (t + 1.0)


def _sc_body(x_hbm, o_hbm, ib0, ib1, ob0, ob1, is0, is1, os0, os1):
    wid = lax.axis_index("s") * 2 + lax.axis_index("c")
    col = pl.multiple_of(wid * _W, 128)
    ibufs, obufs = (ib0, ib1), (ob0, ob1)
    isems, osems = (is0, is1), (os0, os1)

    def in_cp(g, b):
        r0 = pl.multiple_of(g * 8, 8)
        return pltpu.make_async_copy(
            x_hbm.at[pl.ds(r0, 8), pl.ds(col, _W)], ibufs[b], isems[b])

    def out_cp(g, b):
        r0 = pl.multiple_of(g * 8, 8)
        return pltpu.make_async_copy(
            obufs[b], o_hbm.at[pl.ds(r0, 8), pl.ds(col, _W)], osems[b])

    def compute(b):
        @plsc.parallel_loop(0, _W // _L, 1, unroll=4)
        def _(j, b=b):
            c = pl.multiple_of(j * _L, _L)
            for r in range(8):
                obufs[b][r, pl.ds(c, _L)] = _tanh16(
                    ibufs[b][r, pl.ds(c, _L)])

    in_cp(0, 0).start()
    in_cp(1, 1).start()

    def step(s, _):
        for b in range(2):
            g = 2 * s + b
            in_cp(g, b).wait()

            @pl.when(s > 0)
            def _():
                out_cp(g - 2, b).wait()

            compute(b)
            out_cp(g, b).start()

            @pl.when(s < _G // 2 - 1)
            def _():
                in_cp(g + 2, b).start()
        return 0

    lax.fori_loop(0, _G // 2, step, 0, unroll=False)
    out_cp(_G - 2, 0).wait()
    out_cp(_G - 1, 1).wait()


def _tc_body(o_in_hbm, x_hbm, o_hbm, *scratch):
    del o_in_hbm
    ibufs = scratch[:_TC_RING]
    obufs = scratch[_TC_RING:2 * _TC_RING]
    tbuf = scratch[2 * _TC_RING]
    isems = scratch[2 * _TC_RING + 1]
    osems = scratch[2 * _TC_RING + 2]
    tsem_i = scratch[2 * _TC_RING + 3]
    tsem_o = scratch[2 * _TC_RING + 4]
    nchunk = (o_hbm.shape[0] - _SCROWS) // 8

    def in_cp(c, s):
        return pltpu.make_async_copy(
            x_hbm.at[pl.ds(_SCROWS + c * 8, 8), :], ibufs[s], isems.at[s])

    def out_cp(c, s):
        return pltpu.make_async_copy(
            obufs[s], o_hbm.at[pl.ds(_SCROWS + c * 8, 8), :], osems.at[s])

    def tail_in():
        return pltpu.make_async_copy(
            x_hbm.at[pl.ds(0, _SCROWS), pl.ds(_MAIN, _REM)], tbuf, tsem_i)

    def tail_out():
        return pltpu.make_async_copy(
            tbuf, o_hbm.at[pl.ds(0, _SCROWS), pl.ds(_MAIN, _REM)], tsem_o)

    tail_in().start()
    for c in range(min(_TC_RING, nchunk)):
        in_cp(c, c).start()
    for c in range(nchunk):
        s = c % _TC_RING
        in_cp(c, s).wait()
        if c >= _TC_RING:
            out_cp(c - _TC_RING, s).wait()
        obufs[s][...] = jnp.tanh(ibufs[s][...])
        out_cp(c, s).start()
        nc = c + _TC_RING
        if nc < nchunk:
            in_cp(nc, s).start()
    tail_in().wait()
    tbuf[...] = jnp.tanh(tbuf[...])
    tail_out().start()
    tail_out().wait()
    for c in range(max(nchunk - _TC_RING, 0), nchunk):
        out_cp(c, c % _TC_RING).wait()


def kernel(mean, stddev):
    del stddev
    m, n = mean.shape
    mesh = plsc.VectorSubcoreMesh(core_axis_name="c", subcore_axis_name="s")
    sc = pl.kernel(
        _sc_body,
        out_type=jax.ShapeDtypeStruct((m, n), jnp.float32),
        mesh=mesh,
        scratch_types=(
            [pltpu.VMEM((8, _W), jnp.float32) for _ in range(4)]
            + [pltpu.SemaphoreType.DMA for _ in range(4)]
        ),
    )
    sc_out = sc(mean)

    return pl.pallas_call(
        _tc_body,
        in_specs=[pl.BlockSpec(memory_space=pl.ANY),
                  pl.BlockSpec(memory_space=pl.ANY)],
        out_specs=pl.BlockSpec(memory_space=pl.ANY),
        out_shape=jax.ShapeDtypeStruct((m, n), jnp.float32),
        input_output_aliases={0: 0},
        scratch_shapes=(
            [pltpu.VMEM((8, n), jnp.float32) for _ in range(2 * _TC_RING)]
            + [pltpu.VMEM((_SCROWS, _REM), jnp.float32)]
            + [pltpu.SemaphoreType.DMA((_TC_RING,)),
               pltpu.SemaphoreType.DMA((_TC_RING,)),
               pltpu.SemaphoreType.DMA,
               pltpu.SemaphoreType.DMA]
        ),
    )(sc_out, mean)


# speedup vs baseline: 1.1332x; 1.0112x over previous
import jax
import jax.numpy as jnp
from jax import lax
from jax.experimental import pallas as pl
from jax.experimental.pallas import tpu as pltpu
from jax.experimental.pallas import tpu_sc as plsc

_L = 16
_W = 3072
_NW = 32
_G = 6
_SCROWS = 8 * _G
_MAIN = _NW * _W
_REM = 1696
_TC_RING = 4


def _tanh16(v):
    t = jnp.exp(v + v)
    return 1.0 - 2.0 / (t + 1.0)


def _sc_body(x_hbm, o_hbm, ib0, ib1, ob0, ob1, is0, is1, os0, os1):
    wid = lax.axis_index("s") * 2 + lax.axis_index("c")
    col = pl.multiple_of(wid * _W, 128)
    ibufs, obufs = (ib0, ib1), (ob0, ob1)
    isems, osems = (is0, is1), (os0, os1)

    def in_cp(g, b):
        r0 = pl.multiple_of(g * 8, 8)
        return pltpu.make_async_copy(
            x_hbm.at[pl.ds(r0, 8), pl.ds(col, _W)], ibufs[b], isems[b])

    def out_cp(g, b):
        r0 = pl.multiple_of(g * 8, 8)
        return pltpu.make_async_copy(
            obufs[b], o_hbm.at[pl.ds(r0, 8), pl.ds(col, _W)], osems[b])

    def compute(b):
        @plsc.parallel_loop(0, _W // _L, 1, unroll=4)
        def _(j, b=b):
            c = pl.multiple_of(j * _L, _L)
            for r in range(8):
                obufs[b][r, pl.ds(c, _L)] = _tanh16(
                    ibufs[b][r, pl.ds(c, _L)])

    in_cp(0, 0).start()
    in_cp(1, 1).start()

    def step(s, _):
        for b in range(2):
            g = 2 * s + b
            in_cp(g, b).wait()

            @pl.when(s > 0)
            def _():
                out_cp(g - 2, b).wait()

            compute(b)
            out_cp(g, b).start()

            @pl.when(s < _G // 2 - 1)
            def _():
                in_cp(g + 2, b).start()
        return 0

    lax.fori_loop(0, _G // 2, step, 0, unroll=False)
    out_cp(_G - 2, 0).wait()
    out_cp(_G - 1, 1).wait()


def _tc_body(o_in_hbm, x_hbm, o_hbm, *scratch):
    del o_in_hbm
    ibufs = scratch[:_TC_RING]
    obufs = scratch[_TC_RING:2 * _TC_RING]
    tbuf = scratch[2 * _TC_RING]
    isems = scratch[2 * _TC_RING + 1]
    osems = scratch[2 * _TC_RING + 2]
    tsem_i = scratch[2 * _TC_RING + 3]
    tsem_o = scratch[2 * _TC_RING + 4]
    nchunk = (o_hbm.shape[0] - _SCROWS) // 8

    def in_cp(c, s):
        return pltpu.make_async_copy(
            x_hbm.at[pl.ds(_SCROWS + c * 8, 8), :], ibufs[s], isems.at[s])

    def out_cp(c, s):
        return pltpu.make_async_copy(
            obufs[s], o_hbm.at[pl.ds(_SCROWS + c * 8, 8), :], osems.at[s])

    def tail_in():
        return pltpu.make_async_copy(
            x_hbm.at[pl.ds(0, _SCROWS), pl.ds(_MAIN, _REM)], tbuf, tsem_i)

    def tail_out():
        return pltpu.make_async_copy(
            tbuf, o_hbm.at[pl.ds(0, _SCROWS), pl.ds(_MAIN, _REM)], tsem_o)

    tail_in().start()
    for c in range(min(_TC_RING, nchunk)):
        in_cp(c, c).start()
    for c in range(nchunk):
        s = c % _TC_RING
        in_cp(c, s).wait()
        if c >= _TC_RING:
            out_cp(c - _TC_RING, s).wait()
        obufs[s][...] = jnp.tanh(ibufs[s][...])
        out_cp(c, s).start()
        nc = c + _TC_RING
        if nc < nchunk:
            in_cp(nc, s).start()
    tail_in().wait()
    tbuf[...] = jnp.tanh(tbuf[...])
    tail_out().start()
    tail_out().wait()
    for c in range(max(nchunk - _TC_RING, 0), nchunk):
        out_cp(c, c % _TC_RING).wait()


def kernel(mean, stddev):
    del stddev
    m, n = mean.shape
    mesh = plsc.VectorSubcoreMesh(core_axis_name="c", subcore_axis_name="s")
    sc = pl.kernel(
        _sc_body,
        out_type=jax.ShapeDtypeStruct((m, n), jnp.float32),
        mesh=mesh,
        scratch_types=(
            [pltpu.VMEM((8, _W), jnp.float32) for _ in range(4)]
            + [pltpu.SemaphoreType.DMA for _ in range(4)]
        ),
    )
    sc_out = sc(mean)

    return pl.pallas_call(
        _tc_body,
        in_specs=[pl.BlockSpec(memory_space=pl.ANY),
                  pl.BlockSpec(memory_space=pl.ANY)],
        out_specs=pl.BlockSpec(memory_space=pl.ANY),
        out_shape=jax.ShapeDtypeStruct((m, n), jnp.float32),
        input_output_aliases={0: 0},
        scratch_shapes=(
            [pltpu.VMEM((8, n), jnp.float32) for _ in range(2 * _TC_RING)]
            + [pltpu.VMEM((_SCROWS, _REM), jnp.float32)]
            + [pltpu.SemaphoreType.DMA((_TC_RING,)),
               pltpu.SemaphoreType.DMA((_TC_RING,)),
               pltpu.SemaphoreType.DMA,
               pltpu.SemaphoreType.DMA]
        ),
    )(sc_out, mean)
